# Initial kernel scaffold; baseline (speedup 1.0000x reference)
import functools

import jax
import jax.numpy as jnp
from jax import lax
from jax.experimental import pallas as pl
from jax.experimental.pallas import tpu as pltpu
from jax.experimental.pallas import tpu_sc as plsc

VOCAB = 1000
EMBED = 64
B = 4096
L = 50

NC = 2
NS = 16
NW = NC * NS
TOTAL = B * L
PER_W = TOTAL // NW
CHUNK = 128
NCHUNK = PER_W // CHUNK


def _body(idx_hbm, table_hbm, out_hbm, idx_v, rows_v, gsem, osem):
    wid = lax.axis_index("s") * NC + lax.axis_index("c")
    base = wid * PER_W
    pltpu.sync_copy(idx_hbm.at[wid], idx_v)

    def chunk(j, carry):
        pltpu.async_copy(table_hbm.at[idx_v.at[j]], rows_v, gsem).wait()
        pltpu.async_copy(rows_v, out_hbm.at[pl.ds(base + j * CHUNK, CHUNK)],
                         osem).wait()
        return carry

    lax.fori_loop(0, NCHUNK, chunk, 0)


@functools.partial(
    pl.kernel,
    out_type=jax.ShapeDtypeStruct((TOTAL, EMBED), jnp.float32),
    mesh=plsc.VectorSubcoreMesh(core_axis_name="c", subcore_axis_name="s"),
    scratch_types=[
        pltpu.VMEM((NCHUNK, CHUNK), jnp.int32),
        pltpu.VMEM((CHUNK, EMBED), jnp.float32),
        pltpu.SemaphoreType.DMA,
        pltpu.SemaphoreType.DMA,
    ],
)
def _gather_kernel(idx_hbm, table_hbm, out_hbm, idx_v, rows_v, gsem, osem):
    _body(idx_hbm, table_hbm, out_hbm, idx_v, rows_v, gsem, osem)


def kernel(indices, W):
    idx = indices.reshape(NW, NCHUNK, CHUNK)
    out = _gather_kernel(idx, W)
    return out.reshape(B, L, EMBED)


# speedup vs baseline: 4.6026x; 4.6026x over previous
import functools

import jax
import jax.numpy as jnp
from jax import lax
from jax.experimental import pallas as pl
from jax.experimental.pallas import tpu as pltpu
from jax.experimental.pallas import tpu_sc as plsc

VOCAB = 1000
EMBED = 64
B = 4096
L = 50

NC = 2
NS = 16
NW = NC * NS
TOTAL = B * L
PER_W = TOTAL // NW
CHUNK = 128
NCHUNK = PER_W // CHUNK


def _body(idx_hbm, table_hbm, out_hbm, idx_v, rows_v, gsem, osem):
    wid = lax.axis_index("s") * NC + lax.axis_index("c")
    base = wid * PER_W
    pltpu.sync_copy(idx_hbm.at[wid], idx_v)

    def chunk(j, carry):
        pltpu.async_copy(table_hbm.at[idx_v.at[j]], rows_v, gsem).wait()
        pltpu.async_copy(rows_v, out_hbm.at[pl.ds(base + j * CHUNK, CHUNK)],
                         osem).wait()
        return carry

    lax.fori_loop(0, NCHUNK, chunk, 0)


@functools.partial(
    pl.kernel,
    out_type=jax.ShapeDtypeStruct((TOTAL, EMBED), jnp.float32),
    mesh=plsc.VectorSubcoreMesh(core_axis_name="c", subcore_axis_name="s"),
    scratch_types=[
        pltpu.VMEM((NCHUNK, CHUNK), jnp.int32),
        pltpu.VMEM((CHUNK, EMBED), jnp.float32),
        pltpu.SemaphoreType.DMA,
        pltpu.SemaphoreType.DMA,
    ],
    compiler_params=pltpu.CompilerParams(use_tc_tiling_on_sc=False),
)
def _gather_kernel(idx_hbm, table_hbm, out_hbm, idx_v, rows_v, gsem, osem):
    _body(idx_hbm, table_hbm, out_hbm, idx_v, rows_v, gsem, osem)


def kernel(indices, W):
    idx = indices.reshape(NW, NCHUNK, CHUNK)
    out = _gather_kernel(idx, W)
    return out.reshape(B, L, EMBED)


# speedup vs baseline: 4.7686x; 1.0361x over previous
import functools

import jax
import jax.numpy as jnp
from jax import lax
from jax.experimental import pallas as pl
from jax.experimental.pallas import tpu as pltpu
from jax.experimental.pallas import tpu_sc as plsc

VOCAB = 1000
EMBED = 64
B = 4096
L = 50

NC = 2
NS = 16
NW = NC * NS
TOTAL = B * L
PER_W = TOTAL // NW
CHUNK = 128
NCHUNK = PER_W // CHUNK
NBUF = 5
NROUND = NCHUNK // NBUF


def _body(idx_hbm, table_hbm, out_hbm, idx_v, bufs, gsems, osems):
    wid = lax.axis_index("s") * NC + lax.axis_index("c")
    base = wid * PER_W
    pltpu.sync_copy(idx_hbm.at[wid], idx_v)

    def gstart(j, b):
        pltpu.async_copy(table_hbm.at[idx_v.at[j]], bufs[b], gsems[b])

    def gwait(j, b):
        pltpu.make_async_copy(table_hbm.at[idx_v.at[j]], bufs[b],
                              gsems[b]).wait()

    def ostart(j, b):
        pltpu.async_copy(bufs[b],
                         out_hbm.at[pl.ds(base + j * CHUNK, CHUNK)], osems[b])

    def owait(j, b):
        pltpu.make_async_copy(bufs[b],
                              out_hbm.at[pl.ds(base + j * CHUNK, CHUNK)],
                              osems[b]).wait()

    for b in range(NBUF):
        gstart(b, b)

    def round_fn(r, carry):
        g = r * NBUF
        for b in range(NBUF):
            gwait(g + b, b)
            ostart(g + b, b)

        @pl.when(r < NROUND - 1)
        def _():
            for b in range(NBUF):
                owait(g + b, b)
                gstart(g + NBUF + b, b)

        return carry

    lax.fori_loop(0, NROUND, round_fn, 0)

    last = NCHUNK - NBUF
    for b in range(NBUF):
        owait(last + b, b)


@functools.partial(
    pl.kernel,
    out_type=jax.ShapeDtypeStruct((TOTAL, EMBED), jnp.float32),
    mesh=plsc.VectorSubcoreMesh(core_axis_name="c", subcore_axis_name="s"),
    scratch_types=[
        pltpu.VMEM((NCHUNK, CHUNK), jnp.int32),
        [pltpu.VMEM((CHUNK, EMBED), jnp.float32) for _ in range(NBUF)],
        [pltpu.SemaphoreType.DMA for _ in range(NBUF)],
        [pltpu.SemaphoreType.DMA for _ in range(NBUF)],
    ],
    compiler_params=pltpu.CompilerParams(use_tc_tiling_on_sc=False),
)
def _gather_kernel(idx_hbm, table_hbm, out_hbm, idx_v, bufs, gsems, osems):
    _body(idx_hbm, table_hbm, out_hbm, idx_v, bufs, gsems, osems)


def kernel(indices, W):
    idx = indices.reshape(NW, NCHUNK, CHUNK)
    out = _gather_kernel(idx, W)
    return out.reshape(B, L, EMBED)


# speedup vs baseline: 4.7881x; 1.0041x over previous
import functools

import jax
import jax.numpy as jnp
from jax import lax
from jax.experimental import pallas as pl
from jax.experimental.pallas import tpu as pltpu
from jax.experimental.pallas import tpu_sc as plsc

VOCAB = 1000
EMBED = 64
B = 4096
L = 50

NC = 2
NS = 16
NW = NC * NS
NB_W = B // NW
NBUF = 8
NROUND = NB_W // NBUF


def _body(idx_hbm, table_hbm, out_hbm, idx_v, bufs, gsems, osems):
    wid = lax.axis_index("s") * NC + lax.axis_index("c")
    b0 = wid * NB_W
    pltpu.sync_copy(idx_hbm.at[pl.ds(b0, NB_W)], idx_v)

    def gstart(i, k):
        pltpu.async_copy(table_hbm.at[idx_v.at[i]], bufs[k], gsems[k])

    def gwait(i, k):
        pltpu.make_async_copy(table_hbm.at[idx_v.at[i]], bufs[k],
                              gsems[k]).wait()

    def ostart(i, k):
        pltpu.async_copy(bufs[k], out_hbm.at[b0 + i], osems[k])

    def owait(i, k):
        pltpu.make_async_copy(bufs[k], out_hbm.at[b0 + i], osems[k]).wait()

    for k in range(NBUF):
        gstart(k, k)

    def round_fn(r, carry):
        g = r * NBUF
        for k in range(NBUF):
            gwait(g + k, k)
            ostart(g + k, k)

        @pl.when(r < NROUND - 1)
        def _():
            for k in range(NBUF):
                owait(g + k, k)
                gstart(g + NBUF + k, k)

        return carry

    lax.fori_loop(0, NROUND, round_fn, 0)

    last = NB_W - NBUF
    for k in range(NBUF):
        owait(last + k, k)


@functools.partial(
    pl.kernel,
    out_type=jax.ShapeDtypeStruct((B, L, EMBED), jnp.float32),
    mesh=plsc.VectorSubcoreMesh(core_axis_name="c", subcore_axis_name="s"),
    scratch_types=[
        pltpu.VMEM((NB_W, L), jnp.int32),
        [pltpu.VMEM((L, EMBED), jnp.float32) for _ in range(NBUF)],
        [pltpu.SemaphoreType.DMA for _ in range(NBUF)],
        [pltpu.SemaphoreType.DMA for _ in range(NBUF)],
    ],
    compiler_params=pltpu.CompilerParams(use_tc_tiling_on_sc=False),
)
def _gather_kernel(idx_hbm, table_hbm, out_hbm, idx_v, bufs, gsems, osems):
    _body(idx_hbm, table_hbm, out_hbm, idx_v, bufs, gsems, osems)


def kernel(indices, W):
    return _gather_kernel(indices, W)


# speedup vs baseline: 7.9193x; 1.6540x over previous
import functools

import jax
import jax.numpy as jnp
from jax import lax
from jax.experimental import pallas as pl
from jax.experimental.pallas import tpu as pltpu
from jax.experimental.pallas import tpu_sc as plsc

VOCAB = 1000
EMBED = 64
B = 4096
L = 50
VPAD = 1024

NC = 2
NS = 16
NW = NC * NS
BW = B // NW
LANES = 16


def _body(idx_hbm, wt_hbm, out_hbm, table_v, idx_v, buf0, buf1, sem0, sem1):
    wid = lax.axis_index("s") * NC + lax.axis_index("c")
    b0 = wid * BW
    pltpu.sync_copy(wt_hbm, table_v)
    pltpu.sync_copy(idx_hbm.at[:, pl.ds(b0, BW)], idx_v)

    bufs = (buf0, buf1)
    sems = (sem0, sem1)

    def owait(k):
        pltpu.make_async_copy(
            bufs[k], out_hbm.at[pl.ds(0, EMBED), pl.ds(b0, BW)], sems[k]
        ).wait()

    def ostart(l, k):
        pltpu.async_copy(
            bufs[k], out_hbm.at[pl.ds(l * EMBED, EMBED), pl.ds(b0, BW)],
            sems[k],
        )

    def fill(l, buf):
        for g in range(BW // LANES):
            idx16 = idx_v[l, pl.ds(g * LANES, LANES)]

            def e_block(e8, carry):
                base = carry + idx16
                for j in range(8):
                    addr = base + (j * VPAD)
                    v = plsc.load_gather(table_v, [addr])
                    buf[e8 * 8 + j, pl.ds(g * LANES, LANES)] = v
                return carry + (8 * VPAD)

            lax.fori_loop(0, EMBED // 8, e_block, jnp.int32(0))

    def pair(r, carry):
        for k in range(2):
            l = 2 * r + k

            @pl.when(r > 0)
            def _():
                owait(k)

            fill(l, bufs[k])
            ostart(l, k)
        return carry

    lax.fori_loop(0, L // 2, pair, 0)
    owait(0)
    owait(1)


@functools.partial(
    pl.kernel,
    out_type=jax.ShapeDtypeStruct((L * EMBED, B), jnp.float32),
    mesh=plsc.VectorSubcoreMesh(core_axis_name="c", subcore_axis_name="s"),
    scratch_types=[
        pltpu.VMEM((EMBED * VPAD,), jnp.float32),
        pltpu.VMEM((L, BW), jnp.int32),
        pltpu.VMEM((EMBED, BW), jnp.float32),
        pltpu.VMEM((EMBED, BW), jnp.float32),
        pltpu.SemaphoreType.DMA,
        pltpu.SemaphoreType.DMA,
    ],
    compiler_params=pltpu.CompilerParams(use_tc_tiling_on_sc=True,
                                         needs_layout_passes=False),
)
def _gather_kernel(idx_hbm, wt_hbm, out_hbm, table_v, idx_v, buf0, buf1,
                   sem0, sem1):
    _body(idx_hbm, wt_hbm, out_hbm, table_v, idx_v, buf0, buf1, sem0, sem1)


def kernel(indices, W):
    idx_t = indices.T
    wt = jnp.pad(W.T, ((0, 0), (0, VPAD - VOCAB)))
    wt_flat = wt.reshape(EMBED * VPAD)
    out2 = _gather_kernel(idx_t, wt_flat)
    return out2.reshape(L, EMBED, B).transpose(2, 0, 1)
